# Initial kernel scaffold; baseline (speedup 1.0000x reference)
import jax
import jax.numpy as jnp
from jax.experimental import pallas as pl


def kernel(x, edge_index, W, b, gamma, beta):
    raise NotImplementedError("write your pallas kernel here")


# speedup vs baseline: 18.7311x; 18.7311x over previous
import functools

import jax
import jax.numpy as jnp
from jax import lax
from jax.experimental import pallas as pl
from jax.experimental.pallas import tpu as pltpu
from jax.experimental.pallas import tpu_sc as plsc

NC = 2
NS = 16
NW = NC * NS

CHUNK = 80


def _deg_body(n_pad, chunks_per_tile, col_ref, out_ref, cidx, ones_v, zbuf,
              acc):
    c = lax.axis_index("c")
    s = lax.axis_index("s")
    wid = c * NS + s
    rows_per_tile = n_pad // NS

    def fill(i, _):
        ones_v[i] = jnp.ones((16,), jnp.float32)
        return 0
    lax.fori_loop(0, CHUNK, fill, 0)

    def zfill(i, _):
        zbuf[i] = jnp.zeros((16,), jnp.float32)
        return 0
    lax.fori_loop(0, rows_per_tile, zfill, 0)

    pltpu.sync_copy(zbuf, acc.at[pl.ds(s * rows_per_tile, rows_per_tile)])
    plsc.subcore_barrier()

    pltpu.sync_copy(col_ref.at[wid], cidx)

    def step(j, _):
        pltpu.sync_copy(ones_v, acc.at[cidx.at[j]], add=True)
        return 0
    lax.fori_loop(0, chunks_per_tile, step, 0)

    plsc.subcore_barrier()
    pltpu.sync_copy(acc.at[pl.ds(s * rows_per_tile, rows_per_tile)], zbuf)
    pltpu.sync_copy(zbuf,
                    out_ref.at[c, pl.ds(s * rows_per_tile, rows_per_tile)])


def _agg_body(n_pad, dh, chunks_per_tile, y0_ref, y1_ref, row_ref, col_ref,
              out0_ref, out1_ref, ridx, cidx, rows_v, zbuf, acc):
    c = lax.axis_index("c")
    s = lax.axis_index("s")
    wid = c * NS + s
    rows_per_tile = n_pad // NS
    zrows = rows_per_tile // 5

    def zfill(i, _):
        for p in range(dh // 16):
            zbuf[i, pl.ds(p * 16, 16)] = jnp.zeros((16,), jnp.float32)
        return 0

    pltpu.sync_copy(row_ref.at[wid], ridx)
    pltpu.sync_copy(col_ref.at[wid], cidx)

    for y_ref, out_ref in ((y0_ref, out0_ref), (y1_ref, out1_ref)):
        lax.fori_loop(0, zrows, zfill, 0)
        for jj in range(5):
            pltpu.sync_copy(
                zbuf, acc.at[pl.ds(s * rows_per_tile + jj * zrows, zrows)])
        plsc.subcore_barrier()

        def step(j, _):
            pltpu.sync_copy(y_ref.at[ridx.at[j]], rows_v)
            pltpu.sync_copy(rows_v, acc.at[cidx.at[j]], add=True)
            return 0
        lax.fori_loop(0, chunks_per_tile, step, 0)

        plsc.subcore_barrier()
        for jj in range(5):
            base = s * rows_per_tile + jj * zrows
            pltpu.sync_copy(acc.at[pl.ds(base, zrows)], zbuf)
            pltpu.sync_copy(zbuf, out_ref.at[c, pl.ds(base, zrows)])


def _prescale_body(rb, dh, d0_ref, d1_ref, x_ref, y0_ref, y1_ref):
    i = pl.program_id(0)
    deg = d0_ref[pl.ds(i * rb, rb), :] + d1_ref[pl.ds(i * rb, rb), :] + 1.0
    dinv = lax.rsqrt(deg)
    y0_ref[...] = x_ref[:, pl.ds(0, dh)] * dinv
    y1_ref[...] = x_ref[:, pl.ds(dh, dh)] * dinv


def _final_body(rb, s00_ref, s01_ref, s10_ref, s11_ref, y0_ref, y1_ref,
                d0_ref, d1_ref, wt_ref, b_ref, g_ref, beta_ref, o_ref):
    i = pl.program_id(0)
    deg = d0_ref[pl.ds(i * rb, rb), :] + d1_ref[pl.ds(i * rb, rb), :] + 1.0
    dinv = lax.rsqrt(deg)
    dh = y0_ref.shape[1]
    agg_l = (s00_ref[...] + s10_ref[...] + y0_ref[...]) * dinv
    agg_r = (s01_ref[...] + s11_ref[...] + y1_ref[...]) * dinv
    h = jnp.dot(agg_l, wt_ref[pl.ds(0, dh), :],
                preferred_element_type=jnp.float32)
    h = h + jnp.dot(agg_r, wt_ref[pl.ds(dh, dh), :],
                    preferred_element_type=jnp.float32)
    h = h + b_ref[...]
    mean = jnp.mean(h, axis=1, keepdims=True)
    zc = h - mean
    var = jnp.mean(zc * zc, axis=1, keepdims=True)
    o_ref[...] = zc * lax.rsqrt(var + 1e-5) * g_ref[...] + beta_ref[...]


@jax.jit
def kernel(x, edge_index, W, b, gamma, beta):
    n, d = x.shape
    dh = d // 2
    e = edge_index.shape[1]
    nchunks = e // CHUNK
    chunks_per_tile = nchunks // NW

    row3d = edge_index[0].astype(jnp.int32).reshape(NW, chunks_per_tile, CHUNK)
    col3d = edge_index[1].astype(jnp.int32).reshape(NW, chunks_per_tile, CHUNK)

    mesh = plsc.VectorSubcoreMesh(
        core_axis_name="c", subcore_axis_name="s",
        num_cores=NC, num_subcores=NS)

    n_pad = ((n + NS * 40 - 1) // (NS * 40)) * (NS * 40)
    rows_per_tile = n_pad // NS

    deg_call = pl.kernel(
        functools.partial(_deg_body, n_pad, chunks_per_tile),
        out_type=jax.ShapeDtypeStruct((NC, n_pad, 16), jnp.float32),
        mesh=mesh,
        scratch_types=[
            pltpu.VMEM((chunks_per_tile, CHUNK), jnp.int32),
            pltpu.VMEM((CHUNK, 16), jnp.float32),
            pltpu.VMEM((rows_per_tile, 16), jnp.float32),
            pltpu.VMEM_SHARED((n_pad, 16), jnp.float32),
        ],
        compiler_params=pltpu.CompilerParams(use_tc_tiling_on_sc=False),
    )
    degp = deg_call(col3d)
    d0 = lax.slice(degp[0], (0, 0), (n, 1))
    d1 = lax.slice(degp[1], (0, 0), (n, 1))

    nb = 10
    rb = n // nb
    y0, y1 = pl.pallas_call(
        functools.partial(_prescale_body, rb, dh),
        grid=(nb,),
        in_specs=[
            pl.BlockSpec((n, 1), lambda i: (0, 0)),
            pl.BlockSpec((n, 1), lambda i: (0, 0)),
            pl.BlockSpec((rb, d), lambda i: (i, 0)),
        ],
        out_specs=[
            pl.BlockSpec((rb, dh), lambda i: (i, 0)),
            pl.BlockSpec((rb, dh), lambda i: (i, 0)),
        ],
        out_shape=[
            jax.ShapeDtypeStruct((n, dh), jnp.float32),
            jax.ShapeDtypeStruct((n, dh), jnp.float32),
        ],
    )(d0, d1, x)

    agg_call = pl.kernel(
        functools.partial(_agg_body, n_pad, dh, chunks_per_tile),
        out_type=[
            jax.ShapeDtypeStruct((NC, n_pad, dh), jnp.float32),
            jax.ShapeDtypeStruct((NC, n_pad, dh), jnp.float32),
        ],
        mesh=mesh,
        scratch_types=[
            pltpu.VMEM((chunks_per_tile, CHUNK), jnp.int32),
            pltpu.VMEM((chunks_per_tile, CHUNK), jnp.int32),
            pltpu.VMEM((CHUNK, dh), jnp.float32),
            pltpu.VMEM((rows_per_tile // 5, dh), jnp.float32),
            pltpu.VMEM_SHARED((n_pad, dh), jnp.float32),
        ],
        compiler_params=pltpu.CompilerParams(use_tc_tiling_on_sc=False),
    )
    sp0, sp1 = agg_call(y0, y1, row3d, col3d)

    out = pl.pallas_call(
        functools.partial(_final_body, rb),
        grid=(nb,),
        in_specs=[
            pl.BlockSpec((rb, dh), lambda i: (i, 0)),
            pl.BlockSpec((rb, dh), lambda i: (i, 0)),
            pl.BlockSpec((rb, dh), lambda i: (i, 0)),
            pl.BlockSpec((rb, dh), lambda i: (i, 0)),
            pl.BlockSpec((rb, dh), lambda i: (i, 0)),
            pl.BlockSpec((rb, dh), lambda i: (i, 0)),
            pl.BlockSpec((n, 1), lambda i: (0, 0)),
            pl.BlockSpec((n, 1), lambda i: (0, 0)),
            pl.BlockSpec((d, d), lambda i: (0, 0)),
            pl.BlockSpec((1, d), lambda i: (0, 0)),
            pl.BlockSpec((1, d), lambda i: (0, 0)),
            pl.BlockSpec((1, d), lambda i: (0, 0)),
        ],
        out_specs=pl.BlockSpec((rb, d), lambda i: (i, 0)),
        out_shape=jax.ShapeDtypeStruct((n, d), jnp.float32),
    )(lax.slice(sp0[0], (0, 0), (n, dh)),
      lax.slice(sp1[0], (0, 0), (n, dh)),
      lax.slice(sp0[1], (0, 0), (n, dh)),
      lax.slice(sp1[1], (0, 0), (n, dh)),
      y0, y1, d0, d1, W.T, b.reshape(1, d), gamma.reshape(1, d),
      beta.reshape(1, d))
    return out


# speedup vs baseline: 26.8439x; 1.4331x over previous
import functools

import jax
import jax.numpy as jnp
from jax import lax
from jax.experimental import pallas as pl
from jax.experimental.pallas import tpu as pltpu
from jax.experimental.pallas import tpu_sc as plsc

NC = 2
NS = 16
NW = NC * NS

CHUNK = 80


def _deg_body(n_pad, chunks_per_tile, col_ref, out_ref, cidx, ones_v, zbuf,
              acc):
    c = lax.axis_index("c")
    s = lax.axis_index("s")
    wid = c * NS + s
    rows_per_tile = n_pad // NS

    def fill(i, _):
        ones_v[i] = jnp.ones((16,), jnp.float32)
        return 0
    lax.fori_loop(0, CHUNK, fill, 0)

    def zfill(i, _):
        zbuf[i] = jnp.zeros((16,), jnp.float32)
        return 0
    lax.fori_loop(0, rows_per_tile, zfill, 0)

    pltpu.sync_copy(zbuf, acc.at[pl.ds(s * rows_per_tile, rows_per_tile)])
    plsc.subcore_barrier()

    pltpu.sync_copy(col_ref.at[wid], cidx)

    def step(j, _):
        pltpu.sync_copy(ones_v, acc.at[cidx.at[j]], add=True)
        return 0
    lax.fori_loop(0, chunks_per_tile, step, 0)

    plsc.subcore_barrier()
    pltpu.sync_copy(acc.at[pl.ds(s * rows_per_tile, rows_per_tile)], zbuf)
    pltpu.sync_copy(zbuf,
                    out_ref.at[c, pl.ds(s * rows_per_tile, rows_per_tile)])


def _agg_body(n_pad, dh, chunks_per_tile, y0_ref, y1_ref, row_ref, col_ref,
              out0_ref, out1_ref, ridx, cidx, rows_a, rows_b, zbuf, acc,
              gsem_a, gsem_b):
    c = lax.axis_index("c")
    s = lax.axis_index("s")
    wid = c * NS + s
    rows_per_tile = n_pad // NS
    zrows = rows_per_tile // 5
    bufs = ((rows_a, gsem_a), (rows_b, gsem_b))

    def zfill(i, _):
        for p in range(dh // 16):
            zbuf[i, pl.ds(p * 16, 16)] = jnp.zeros((16,), jnp.float32)
        return 0

    pltpu.sync_copy(row_ref.at[wid], ridx)
    pltpu.sync_copy(col_ref.at[wid], cidx)

    for y_ref, out_ref in ((y0_ref, out0_ref), (y1_ref, out1_ref)):
        lax.fori_loop(0, zrows, zfill, 0)
        for jj in range(5):
            pltpu.sync_copy(
                zbuf, acc.at[pl.ds(s * rows_per_tile + jj * zrows, zrows)])
        plsc.subcore_barrier()

        for b, (buf, sem) in enumerate(bufs):
            pltpu.async_copy(y_ref.at[ridx.at[b]], buf, sem)

        npairs = (chunks_per_tile - 2) // 2
        tail = chunks_per_tile - 2 * npairs

        def pair(k, _):
            for b, (buf, sem) in enumerate(bufs):
                jj = 2 * k + b
                pltpu.make_async_copy(y_ref.at[ridx.at[jj]], buf, sem).wait()
                pltpu.sync_copy(buf, acc.at[cidx.at[jj]], add=True)
                pltpu.async_copy(y_ref.at[ridx.at[jj + 2]], buf, sem)
            return 0
        lax.fori_loop(0, npairs, pair, 0)

        for t in range(tail):
            jj = 2 * npairs + t
            buf, sem = bufs[t % 2]
            pltpu.make_async_copy(y_ref.at[ridx.at[jj]], buf, sem).wait()
            pltpu.sync_copy(buf, acc.at[cidx.at[jj]], add=True)
            if t == 0 and tail == 3:
                pltpu.async_copy(y_ref.at[ridx.at[jj + 2]], buf, sem)

        plsc.subcore_barrier()
        for jj in range(5):
            base = s * rows_per_tile + jj * zrows
            pltpu.sync_copy(acc.at[pl.ds(base, zrows)], zbuf)
            pltpu.sync_copy(zbuf, out_ref.at[c, pl.ds(base, zrows)])


def _prescale_body(rb, dh, d0_ref, d1_ref, x_ref, y0_ref, y1_ref):
    i = pl.program_id(0)
    deg = d0_ref[pl.ds(i * rb, rb), :] + d1_ref[pl.ds(i * rb, rb), :] + 1.0
    dinv = lax.rsqrt(deg)
    y0_ref[...] = x_ref[:, pl.ds(0, dh)] * dinv
    y1_ref[...] = x_ref[:, pl.ds(dh, dh)] * dinv


def _final_body(rb, s00_ref, s01_ref, s10_ref, s11_ref, y0_ref, y1_ref,
                d0_ref, d1_ref, wt_ref, b_ref, g_ref, beta_ref, o_ref):
    i = pl.program_id(0)
    deg = d0_ref[pl.ds(i * rb, rb), :] + d1_ref[pl.ds(i * rb, rb), :] + 1.0
    dinv = lax.rsqrt(deg)
    dh = y0_ref.shape[1]
    agg_l = (s00_ref[...] + s10_ref[...] + y0_ref[...]) * dinv
    agg_r = (s01_ref[...] + s11_ref[...] + y1_ref[...]) * dinv
    h = jnp.dot(agg_l, wt_ref[pl.ds(0, dh), :],
                preferred_element_type=jnp.float32)
    h = h + jnp.dot(agg_r, wt_ref[pl.ds(dh, dh), :],
                    preferred_element_type=jnp.float32)
    h = h + b_ref[...]
    mean = jnp.mean(h, axis=1, keepdims=True)
    zc = h - mean
    var = jnp.mean(zc * zc, axis=1, keepdims=True)
    o_ref[...] = zc * lax.rsqrt(var + 1e-5) * g_ref[...] + beta_ref[...]


@jax.jit
def kernel(x, edge_index, W, b, gamma, beta):
    n, d = x.shape
    dh = d // 2
    e = edge_index.shape[1]
    nchunks = e // CHUNK
    chunks_per_tile = nchunks // NW

    row3d = edge_index[0].astype(jnp.int32).reshape(NW, chunks_per_tile, CHUNK)
    col3d = edge_index[1].astype(jnp.int32).reshape(NW, chunks_per_tile, CHUNK)

    mesh = plsc.VectorSubcoreMesh(
        core_axis_name="c", subcore_axis_name="s",
        num_cores=NC, num_subcores=NS)

    n_pad = ((n + NS * 40 - 1) // (NS * 40)) * (NS * 40)
    rows_per_tile = n_pad // NS

    deg_call = pl.kernel(
        functools.partial(_deg_body, n_pad, chunks_per_tile),
        out_type=jax.ShapeDtypeStruct((NC, n_pad, 16), jnp.float32),
        mesh=mesh,
        scratch_types=[
            pltpu.VMEM((chunks_per_tile, CHUNK), jnp.int32),
            pltpu.VMEM((CHUNK, 16), jnp.float32),
            pltpu.VMEM((rows_per_tile, 16), jnp.float32),
            pltpu.VMEM_SHARED((n_pad, 16), jnp.float32),
        ],
        compiler_params=pltpu.CompilerParams(use_tc_tiling_on_sc=False),
    )
    degp = deg_call(col3d)
    d0 = lax.slice(degp[0], (0, 0), (n, 1))
    d1 = lax.slice(degp[1], (0, 0), (n, 1))

    nb = 10
    rb = n // nb
    y0, y1 = pl.pallas_call(
        functools.partial(_prescale_body, rb, dh),
        grid=(nb,),
        in_specs=[
            pl.BlockSpec((n, 1), lambda i: (0, 0)),
            pl.BlockSpec((n, 1), lambda i: (0, 0)),
            pl.BlockSpec((rb, d), lambda i: (i, 0)),
        ],
        out_specs=[
            pl.BlockSpec((rb, dh), lambda i: (i, 0)),
            pl.BlockSpec((rb, dh), lambda i: (i, 0)),
        ],
        out_shape=[
            jax.ShapeDtypeStruct((n, dh), jnp.float32),
            jax.ShapeDtypeStruct((n, dh), jnp.float32),
        ],
    )(d0, d1, x)

    agg_call = pl.kernel(
        functools.partial(_agg_body, n_pad, dh, chunks_per_tile),
        out_type=[
            jax.ShapeDtypeStruct((NC, n_pad, dh), jnp.float32),
            jax.ShapeDtypeStruct((NC, n_pad, dh), jnp.float32),
        ],
        mesh=mesh,
        scratch_types=[
            pltpu.VMEM((chunks_per_tile, CHUNK), jnp.int32),
            pltpu.VMEM((chunks_per_tile, CHUNK), jnp.int32),
            pltpu.VMEM((CHUNK, dh), jnp.float32),
            pltpu.VMEM((CHUNK, dh), jnp.float32),
            pltpu.VMEM((rows_per_tile // 5, dh), jnp.float32),
            pltpu.VMEM_SHARED((n_pad, dh), jnp.float32),
            pltpu.SemaphoreType.DMA,
            pltpu.SemaphoreType.DMA,
        ],
        compiler_params=pltpu.CompilerParams(use_tc_tiling_on_sc=False),
    )
    sp0, sp1 = agg_call(y0, y1, row3d, col3d)

    out = pl.pallas_call(
        functools.partial(_final_body, rb),
        grid=(nb,),
        in_specs=[
            pl.BlockSpec((rb, dh), lambda i: (i, 0)),
            pl.BlockSpec((rb, dh), lambda i: (i, 0)),
            pl.BlockSpec((rb, dh), lambda i: (i, 0)),
            pl.BlockSpec((rb, dh), lambda i: (i, 0)),
            pl.BlockSpec((rb, dh), lambda i: (i, 0)),
            pl.BlockSpec((rb, dh), lambda i: (i, 0)),
            pl.BlockSpec((n, 1), lambda i: (0, 0)),
            pl.BlockSpec((n, 1), lambda i: (0, 0)),
            pl.BlockSpec((d, d), lambda i: (0, 0)),
            pl.BlockSpec((1, d), lambda i: (0, 0)),
            pl.BlockSpec((1, d), lambda i: (0, 0)),
            pl.BlockSpec((1, d), lambda i: (0, 0)),
        ],
        out_specs=pl.BlockSpec((rb, d), lambda i: (i, 0)),
        out_shape=jax.ShapeDtypeStruct((n, d), jnp.float32),
    )(lax.slice(sp0[0], (0, 0), (n, dh)),
      lax.slice(sp1[0], (0, 0), (n, dh)),
      lax.slice(sp0[1], (0, 0), (n, dh)),
      lax.slice(sp1[1], (0, 0), (n, dh)),
      y0, y1, d0, d1, W.T, b.reshape(1, d), gamma.reshape(1, d),
      beta.reshape(1, d))
    return out


# speedup vs baseline: 32.1799x; 1.1988x over previous
import functools

import jax
import jax.numpy as jnp
from jax import lax
from jax.experimental import pallas as pl
from jax.experimental.pallas import tpu as pltpu
from jax.experimental.pallas import tpu_sc as plsc

NC = 2
NS = 16
NW = NC * NS

CHUNK = 80


def _deg_body(n_pad, chunks_per_tile, col_ref, out_ref, cidx, ones_v, zbuf,
              acc, dsem):
    c = lax.axis_index("c")
    s = lax.axis_index("s")
    wid = c * NS + s
    rows_per_tile = n_pad // NS

    def fill(i, _):
        ones_v[i] = jnp.ones((16,), jnp.float32)
        return 0
    lax.fori_loop(0, CHUNK, fill, 0)

    def zfill(i, _):
        zbuf[i] = jnp.zeros((16,), jnp.float32)
        return 0
    lax.fori_loop(0, rows_per_tile, zfill, 0)

    pltpu.sync_copy(zbuf, acc.at[pl.ds(s * rows_per_tile, rows_per_tile)])
    plsc.subcore_barrier()

    pltpu.sync_copy(col_ref.at[wid], cidx)

    def step(j, _):
        pltpu.async_copy(ones_v, acc.at[cidx.at[j]], dsem, add=True)
        return 0
    lax.fori_loop(0, chunks_per_tile, step, 0)

    def drain(j, _):
        pltpu.make_async_copy(ones_v, acc.at[cidx.at[0]], dsem).wait()
        return 0
    lax.fori_loop(0, chunks_per_tile, drain, 0)

    plsc.subcore_barrier()
    pltpu.sync_copy(acc.at[pl.ds(s * rows_per_tile, rows_per_tile)], zbuf)
    pltpu.sync_copy(zbuf,
                    out_ref.at[c, pl.ds(s * rows_per_tile, rows_per_tile)])


def _agg_body(n_pad, dh, chunks_per_tile, nbuf, y0_ref, y1_ref, row_ref,
              col_ref, out0_ref, out1_ref, ridx, cidx, *scratch):
    rows = scratch[:nbuf]
    zbuf = scratch[nbuf]
    acc = scratch[nbuf + 1]
    gsem = scratch[nbuf + 2:2 * nbuf + 2]
    ssem = scratch[2 * nbuf + 2:]
    c = lax.axis_index("c")
    s = lax.axis_index("s")
    wid = c * NS + s
    rows_per_tile = n_pad // NS
    zrows = rows_per_tile // 5
    ngroups = chunks_per_tile // nbuf

    def zfill(i, _):
        for p in range(dh // 16):
            zbuf[i, pl.ds(p * 16, 16)] = jnp.zeros((16,), jnp.float32)
        return 0

    pltpu.sync_copy(row_ref.at[wid], ridx)
    pltpu.sync_copy(col_ref.at[wid], cidx)

    for y_ref, out_ref in ((y0_ref, out0_ref), (y1_ref, out1_ref)):
        lax.fori_loop(0, zrows, zfill, 0)
        for jj in range(5):
            pltpu.sync_copy(
                zbuf, acc.at[pl.ds(s * rows_per_tile + jj * zrows, zrows)])
        plsc.subcore_barrier()

        for b in range(nbuf):
            pltpu.async_copy(y_ref.at[ridx.at[b]], rows[b], gsem[b])

        def group(k, _):
            for b in range(nbuf):
                jj = nbuf * k + b
                pltpu.make_async_copy(y_ref.at[ridx.at[jj]], rows[b],
                                      gsem[b]).wait()
                pltpu.async_copy(rows[b], acc.at[cidx.at[jj]], ssem[b],
                                 add=True)
            for b in range(nbuf):
                jj = nbuf * k + b
                pltpu.make_async_copy(rows[b], acc.at[cidx.at[0]],
                                      ssem[b]).wait()
                pltpu.async_copy(y_ref.at[ridx.at[jj + nbuf]], rows[b],
                                 gsem[b])
            return 0
        lax.fori_loop(0, ngroups - 1, group, 0)

        for b in range(nbuf):
            jj = nbuf * (ngroups - 1) + b
            pltpu.make_async_copy(y_ref.at[ridx.at[jj]], rows[b],
                                  gsem[b]).wait()
            pltpu.async_copy(rows[b], acc.at[cidx.at[jj]], ssem[b], add=True)
        for b in range(nbuf):
            pltpu.make_async_copy(rows[b], acc.at[cidx.at[0]], ssem[b]).wait()

        plsc.subcore_barrier()
        for jj in range(5):
            base = s * rows_per_tile + jj * zrows
            pltpu.sync_copy(acc.at[pl.ds(base, zrows)], zbuf)
            pltpu.sync_copy(zbuf, out_ref.at[c, pl.ds(base, zrows)])


def _prescale_body(rb, dh, d0_ref, d1_ref, x_ref, y0_ref, y1_ref):
    i = pl.program_id(0)
    deg = d0_ref[pl.ds(i * rb, rb), :] + d1_ref[pl.ds(i * rb, rb), :] + 1.0
    dinv = lax.rsqrt(deg)
    y0_ref[...] = x_ref[:, pl.ds(0, dh)] * dinv
    y1_ref[...] = x_ref[:, pl.ds(dh, dh)] * dinv


def _final_body(rb, s00_ref, s01_ref, s10_ref, s11_ref, y0_ref, y1_ref,
                d0_ref, d1_ref, wt_ref, b_ref, g_ref, beta_ref, o_ref):
    i = pl.program_id(0)
    deg = d0_ref[pl.ds(i * rb, rb), :] + d1_ref[pl.ds(i * rb, rb), :] + 1.0
    dinv = lax.rsqrt(deg)
    dh = y0_ref.shape[1]
    agg_l = (s00_ref[...] + s10_ref[...] + y0_ref[...]) * dinv
    agg_r = (s01_ref[...] + s11_ref[...] + y1_ref[...]) * dinv
    h = jnp.dot(agg_l, wt_ref[pl.ds(0, dh), :],
                preferred_element_type=jnp.float32)
    h = h + jnp.dot(agg_r, wt_ref[pl.ds(dh, dh), :],
                    preferred_element_type=jnp.float32)
    h = h + b_ref[...]
    mean = jnp.mean(h, axis=1, keepdims=True)
    zc = h - mean
    var = jnp.mean(zc * zc, axis=1, keepdims=True)
    o_ref[...] = zc * lax.rsqrt(var + 1e-5) * g_ref[...] + beta_ref[...]


@jax.jit
def kernel(x, edge_index, W, b, gamma, beta):
    n, d = x.shape
    dh = d // 2
    e = edge_index.shape[1]
    nchunks = e // CHUNK
    chunks_per_tile = nchunks // NW

    row3d = edge_index[0].astype(jnp.int32).reshape(NW, chunks_per_tile, CHUNK)
    col3d = edge_index[1].astype(jnp.int32).reshape(NW, chunks_per_tile, CHUNK)

    mesh = plsc.VectorSubcoreMesh(
        core_axis_name="c", subcore_axis_name="s",
        num_cores=NC, num_subcores=NS)

    n_pad = ((n + NS * 40 - 1) // (NS * 40)) * (NS * 40)
    rows_per_tile = n_pad // NS

    deg_call = pl.kernel(
        functools.partial(_deg_body, n_pad, chunks_per_tile),
        out_type=jax.ShapeDtypeStruct((NC, n_pad, 16), jnp.float32),
        mesh=mesh,
        scratch_types=[
            pltpu.VMEM((chunks_per_tile, CHUNK), jnp.int32),
            pltpu.VMEM((CHUNK, 16), jnp.float32),
            pltpu.VMEM((rows_per_tile, 16), jnp.float32),
            pltpu.VMEM_SHARED((n_pad, 16), jnp.float32),
            pltpu.SemaphoreType.DMA,
        ],
        compiler_params=pltpu.CompilerParams(use_tc_tiling_on_sc=False),
    )
    degp = deg_call(col3d)
    d0 = lax.slice(degp[0], (0, 0), (n, 1))
    d1 = lax.slice(degp[1], (0, 0), (n, 1))

    nb = 10
    rb = n // nb
    y0, y1 = pl.pallas_call(
        functools.partial(_prescale_body, rb, dh),
        grid=(nb,),
        in_specs=[
            pl.BlockSpec((n, 1), lambda i: (0, 0)),
            pl.BlockSpec((n, 1), lambda i: (0, 0)),
            pl.BlockSpec((rb, d), lambda i: (i, 0)),
        ],
        out_specs=[
            pl.BlockSpec((rb, dh), lambda i: (i, 0)),
            pl.BlockSpec((rb, dh), lambda i: (i, 0)),
        ],
        out_shape=[
            jax.ShapeDtypeStruct((n, dh), jnp.float32),
            jax.ShapeDtypeStruct((n, dh), jnp.float32),
        ],
    )(d0, d1, x)

    nbuf = 5
    agg_call = pl.kernel(
        functools.partial(_agg_body, n_pad, dh, chunks_per_tile, nbuf),
        out_type=[
            jax.ShapeDtypeStruct((NC, n_pad, dh), jnp.float32),
            jax.ShapeDtypeStruct((NC, n_pad, dh), jnp.float32),
        ],
        mesh=mesh,
        scratch_types=(
            [pltpu.VMEM((chunks_per_tile, CHUNK), jnp.int32),
             pltpu.VMEM((chunks_per_tile, CHUNK), jnp.int32)]
            + [pltpu.VMEM((CHUNK, dh), jnp.float32) for _ in range(nbuf)]
            + [pltpu.VMEM((rows_per_tile // 5, dh), jnp.float32),
               pltpu.VMEM_SHARED((n_pad, dh), jnp.float32)]
            + [pltpu.SemaphoreType.DMA for _ in range(2 * nbuf)]
        ),
        compiler_params=pltpu.CompilerParams(use_tc_tiling_on_sc=False),
    )
    sp0, sp1 = agg_call(y0, y1, row3d, col3d)

    out = pl.pallas_call(
        functools.partial(_final_body, rb),
        grid=(nb,),
        in_specs=[
            pl.BlockSpec((rb, dh), lambda i: (i, 0)),
            pl.BlockSpec((rb, dh), lambda i: (i, 0)),
            pl.BlockSpec((rb, dh), lambda i: (i, 0)),
            pl.BlockSpec((rb, dh), lambda i: (i, 0)),
            pl.BlockSpec((rb, dh), lambda i: (i, 0)),
            pl.BlockSpec((rb, dh), lambda i: (i, 0)),
            pl.BlockSpec((n, 1), lambda i: (0, 0)),
            pl.BlockSpec((n, 1), lambda i: (0, 0)),
            pl.BlockSpec((d, d), lambda i: (0, 0)),
            pl.BlockSpec((1, d), lambda i: (0, 0)),
            pl.BlockSpec((1, d), lambda i: (0, 0)),
            pl.BlockSpec((1, d), lambda i: (0, 0)),
        ],
        out_specs=pl.BlockSpec((rb, d), lambda i: (i, 0)),
        out_shape=jax.ShapeDtypeStruct((n, d), jnp.float32),
    )(lax.slice(sp0[0], (0, 0), (n, dh)),
      lax.slice(sp1[0], (0, 0), (n, dh)),
      lax.slice(sp0[1], (0, 0), (n, dh)),
      lax.slice(sp1[1], (0, 0), (n, dh)),
      y0, y1, d0, d1, W.T, b.reshape(1, d), gamma.reshape(1, d),
      beta.reshape(1, d))
    return out


# speedup vs baseline: 34.0605x; 1.0584x over previous
import functools

import jax
import jax.numpy as jnp
from jax import lax
from jax.experimental import pallas as pl
from jax.experimental.pallas import tpu as pltpu
from jax.experimental.pallas import tpu_sc as plsc

NC = 2
NS = 16
NW = NC * NS

CHUNK = 80


def _deg_body(n_pad, chunks_per_tile, col_ref, out_ref, cidx, ones_v, zbuf,
              acc, dsem):
    c = lax.axis_index("c")
    s = lax.axis_index("s")
    wid = c * NS + s
    rows_per_tile = n_pad // NS

    def fill(i, _):
        ones_v[i] = jnp.ones((16,), jnp.float32)
        return 0
    lax.fori_loop(0, CHUNK, fill, 0)

    def zfill(i, _):
        zbuf[i] = jnp.zeros((16,), jnp.float32)
        return 0
    lax.fori_loop(0, rows_per_tile, zfill, 0)

    pltpu.sync_copy(zbuf, acc.at[pl.ds(s * rows_per_tile, rows_per_tile)])
    plsc.subcore_barrier()

    pltpu.sync_copy(col_ref.at[wid], cidx)

    def step(j, _):
        pltpu.async_copy(ones_v, acc.at[cidx.at[j]], dsem, add=True)
        return 0
    lax.fori_loop(0, chunks_per_tile, step, 0)

    def drain(j, _):
        pltpu.make_async_copy(ones_v, acc.at[cidx.at[0]], dsem).wait()
        return 0
    lax.fori_loop(0, chunks_per_tile, drain, 0)

    plsc.subcore_barrier()
    pltpu.sync_copy(acc.at[pl.ds(s * rows_per_tile, rows_per_tile)], zbuf)
    pltpu.sync_copy(zbuf,
                    out_ref.at[c, pl.ds(s * rows_per_tile, rows_per_tile)])


def _agg_body(n_pad, dh, chunks_per_tile, nbuf, y0_ref, y1_ref, row_ref,
              col_ref, out0_ref, out1_ref, ridx, cidx, *scratch):
    rows = scratch[:nbuf]
    zbuf = scratch[nbuf]
    acc = scratch[nbuf + 1]
    gsem = scratch[nbuf + 2:2 * nbuf + 2]
    ssem = scratch[2 * nbuf + 2:]
    c = lax.axis_index("c")
    s = lax.axis_index("s")
    wid = c * NS + s
    rows_per_tile = n_pad // NS
    zrows = rows_per_tile // 5
    ngroups = chunks_per_tile // nbuf

    def zfill(i, _):
        for p in range(dh // 16):
            zbuf[i, pl.ds(p * 16, 16)] = jnp.zeros((16,), jnp.float32)
        return 0

    pltpu.sync_copy(row_ref.at[wid], ridx)
    pltpu.sync_copy(col_ref.at[wid], cidx)

    for y_ref, out_ref in ((y0_ref, out0_ref), (y1_ref, out1_ref)):
        lax.fori_loop(0, zrows, zfill, 0)
        for jj in range(5):
            pltpu.sync_copy(
                zbuf, acc.at[pl.ds(s * rows_per_tile + jj * zrows, zrows)])
        plsc.subcore_barrier()

        for b in range(nbuf):
            pltpu.async_copy(y_ref.at[ridx.at[b]], rows[b], gsem[b])

        def group(k, _):
            for b in range(nbuf):
                jj = nbuf * k + b
                pltpu.make_async_copy(y_ref.at[ridx.at[jj]], rows[b],
                                      gsem[b]).wait()
                pltpu.async_copy(rows[b], acc.at[cidx.at[jj]], ssem[b],
                                 add=True)
            for b in range(nbuf):
                jj = nbuf * k + b
                pltpu.make_async_copy(rows[b], acc.at[cidx.at[0]],
                                      ssem[b]).wait()
                pltpu.async_copy(y_ref.at[ridx.at[jj + nbuf]], rows[b],
                                 gsem[b])
            return 0
        lax.fori_loop(0, ngroups - 1, group, 0)

        for b in range(nbuf):
            jj = nbuf * (ngroups - 1) + b
            pltpu.make_async_copy(y_ref.at[ridx.at[jj]], rows[b],
                                  gsem[b]).wait()
            pltpu.async_copy(rows[b], acc.at[cidx.at[jj]], ssem[b], add=True)
        for b in range(nbuf):
            pltpu.make_async_copy(rows[b], acc.at[cidx.at[0]], ssem[b]).wait()

        plsc.subcore_barrier()
        for jj in range(5):
            base = s * rows_per_tile + jj * zrows
            pltpu.sync_copy(acc.at[pl.ds(base, zrows)], zbuf)
            pltpu.sync_copy(zbuf, out_ref.at[c, pl.ds(base, zrows)])


def _prescale_body(rb, dh, d0_ref, d1_ref, x_ref, y0_ref, y1_ref):
    i = pl.program_id(0)
    deg = d0_ref[pl.ds(i * rb, rb), :] + d1_ref[pl.ds(i * rb, rb), :] + 1.0
    dinv = lax.rsqrt(deg)
    y0_ref[...] = x_ref[:, pl.ds(0, dh)] * dinv
    y1_ref[...] = x_ref[:, pl.ds(dh, dh)] * dinv


def _final_body(rb, s00_ref, s01_ref, s10_ref, s11_ref, y0_ref, y1_ref,
                d0_ref, d1_ref, wt_ref, b_ref, g_ref, beta_ref, o_ref):
    i = pl.program_id(0)
    deg = d0_ref[pl.ds(i * rb, rb), :] + d1_ref[pl.ds(i * rb, rb), :] + 1.0
    dinv = lax.rsqrt(deg)
    dh = y0_ref.shape[-1]
    agg_l = (s00_ref[0] + s10_ref[0] + y0_ref[...]) * dinv
    agg_r = (s01_ref[0] + s11_ref[0] + y1_ref[...]) * dinv
    h = jnp.dot(agg_l, wt_ref[pl.ds(0, dh), :],
                preferred_element_type=jnp.float32)
    h = h + jnp.dot(agg_r, wt_ref[pl.ds(dh, dh), :],
                    preferred_element_type=jnp.float32)
    h = h + b_ref[...]
    mean = jnp.mean(h, axis=1, keepdims=True)
    zc = h - mean
    var = jnp.mean(zc * zc, axis=1, keepdims=True)
    o_ref[...] = zc * lax.rsqrt(var + 1e-5) * g_ref[...] + beta_ref[...]


@jax.jit
def kernel(x, edge_index, W, b, gamma, beta):
    n, d = x.shape
    dh = d // 2
    e = edge_index.shape[1]
    nchunks = e // CHUNK
    chunks_per_tile = nchunks // NW

    row3d = edge_index[0].astype(jnp.int32).reshape(NW, chunks_per_tile, CHUNK)
    col3d = edge_index[1].astype(jnp.int32).reshape(NW, chunks_per_tile, CHUNK)

    mesh = plsc.VectorSubcoreMesh(
        core_axis_name="c", subcore_axis_name="s",
        num_cores=NC, num_subcores=NS)

    n_pad = ((n + NS * 40 - 1) // (NS * 40)) * (NS * 40)
    rows_per_tile = n_pad // NS

    deg_call = pl.kernel(
        functools.partial(_deg_body, n_pad, chunks_per_tile),
        out_type=jax.ShapeDtypeStruct((NC, n_pad, 16), jnp.float32),
        mesh=mesh,
        scratch_types=[
            pltpu.VMEM((chunks_per_tile, CHUNK), jnp.int32),
            pltpu.VMEM((CHUNK, 16), jnp.float32),
            pltpu.VMEM((rows_per_tile, 16), jnp.float32),
            pltpu.VMEM_SHARED((n_pad, 16), jnp.float32),
            pltpu.SemaphoreType.DMA,
        ],
        compiler_params=pltpu.CompilerParams(use_tc_tiling_on_sc=False),
    )
    degp = deg_call(col3d)
    d0 = lax.slice(degp[0], (0, 0), (n, 1))
    d1 = lax.slice(degp[1], (0, 0), (n, 1))

    nb = 10
    rb = n // nb
    y0, y1 = pl.pallas_call(
        functools.partial(_prescale_body, rb, dh),
        grid=(nb,),
        in_specs=[
            pl.BlockSpec((n, 1), lambda i: (0, 0)),
            pl.BlockSpec((n, 1), lambda i: (0, 0)),
            pl.BlockSpec((rb, d), lambda i: (i, 0)),
        ],
        out_specs=[
            pl.BlockSpec((rb, dh), lambda i: (i, 0)),
            pl.BlockSpec((rb, dh), lambda i: (i, 0)),
        ],
        out_shape=[
            jax.ShapeDtypeStruct((n, dh), jnp.float32),
            jax.ShapeDtypeStruct((n, dh), jnp.float32),
        ],
    )(d0, d1, x)

    nbuf = 5
    agg_call = pl.kernel(
        functools.partial(_agg_body, n_pad, dh, chunks_per_tile, nbuf),
        out_type=[
            jax.ShapeDtypeStruct((NC, n_pad, dh), jnp.float32),
            jax.ShapeDtypeStruct((NC, n_pad, dh), jnp.float32),
        ],
        mesh=mesh,
        scratch_types=(
            [pltpu.VMEM((chunks_per_tile, CHUNK), jnp.int32),
             pltpu.VMEM((chunks_per_tile, CHUNK), jnp.int32)]
            + [pltpu.VMEM((CHUNK, dh), jnp.float32) for _ in range(nbuf)]
            + [pltpu.VMEM((rows_per_tile // 5, dh), jnp.float32),
               pltpu.VMEM_SHARED((n_pad, dh), jnp.float32)]
            + [pltpu.SemaphoreType.DMA for _ in range(2 * nbuf)]
        ),
        compiler_params=pltpu.CompilerParams(use_tc_tiling_on_sc=False),
    )
    sp0, sp1 = agg_call(y0, y1, row3d, col3d)

    out = pl.pallas_call(
        functools.partial(_final_body, rb),
        grid=(nb,),
        in_specs=[
            pl.BlockSpec((1, rb, dh), lambda i: (0, i, 0)),
            pl.BlockSpec((1, rb, dh), lambda i: (0, i, 0)),
            pl.BlockSpec((1, rb, dh), lambda i: (1, i, 0)),
            pl.BlockSpec((1, rb, dh), lambda i: (1, i, 0)),
            pl.BlockSpec((rb, dh), lambda i: (i, 0)),
            pl.BlockSpec((rb, dh), lambda i: (i, 0)),
            pl.BlockSpec((n, 1), lambda i: (0, 0)),
            pl.BlockSpec((n, 1), lambda i: (0, 0)),
            pl.BlockSpec((d, d), lambda i: (0, 0)),
            pl.BlockSpec((1, d), lambda i: (0, 0)),
            pl.BlockSpec((1, d), lambda i: (0, 0)),
            pl.BlockSpec((1, d), lambda i: (0, 0)),
        ],
        out_specs=pl.BlockSpec((rb, d), lambda i: (i, 0)),
        out_shape=jax.ShapeDtypeStruct((n, d), jnp.float32),
    )(sp0, sp1, sp0, sp1,
      y0, y1, d0, d1, W.T, b.reshape(1, d), gamma.reshape(1, d),
      beta.reshape(1, d))
    return out


# speedup vs baseline: 35.4477x; 1.0407x over previous
import functools

import jax
import jax.numpy as jnp
from jax import lax
from jax.experimental import pallas as pl
from jax.experimental.pallas import tpu as pltpu
from jax.experimental.pallas import tpu_sc as plsc

NC = 2
NS = 16
NW = NC * NS

CHUNK = 80


def _deg_body(n_pad, chunks_per_tile, col_ref, out_ref, cidx, ones_v, zbuf,
              acc, dsem):
    c = lax.axis_index("c")
    s = lax.axis_index("s")
    wid = c * NS + s
    rows_per_tile = n_pad // NS

    def fill(i, _):
        ones_v[i] = jnp.ones((16,), jnp.float32)
        return 0
    lax.fori_loop(0, CHUNK, fill, 0)

    def zfill(i, _):
        zbuf[i] = jnp.zeros((16,), jnp.float32)
        return 0
    lax.fori_loop(0, rows_per_tile, zfill, 0)

    pltpu.sync_copy(zbuf, acc.at[pl.ds(s * rows_per_tile, rows_per_tile)])
    plsc.subcore_barrier()

    pltpu.sync_copy(col_ref.at[wid], cidx)

    def step(j, _):
        pltpu.async_copy(ones_v, acc.at[cidx.at[j]], dsem, add=True)
        return 0
    lax.fori_loop(0, chunks_per_tile, step, 0)

    def drain(j, _):
        pltpu.make_async_copy(ones_v, acc.at[cidx.at[0]], dsem).wait()
        return 0
    lax.fori_loop(0, chunks_per_tile, drain, 0)

    plsc.subcore_barrier()
    pltpu.sync_copy(acc.at[pl.ds(s * rows_per_tile, rows_per_tile)], zbuf)
    pltpu.sync_copy(zbuf,
                    out_ref.at[c, pl.ds(s * rows_per_tile, rows_per_tile)])


def _agg_body(n_pad, dh, chunks_per_tile, nbuf, y0_ref, y1_ref, row_ref,
              col_ref, out0_ref, out1_ref, ridx, cidx, *scratch):
    rows = scratch[:nbuf]
    zbuf = scratch[nbuf]
    acc = scratch[nbuf + 1]
    gsem = scratch[nbuf + 2:2 * nbuf + 2]
    ssem = scratch[2 * nbuf + 2:]
    c = lax.axis_index("c")
    s = lax.axis_index("s")
    wid = c * NS + s
    rows_per_tile = n_pad // NS
    zrows = rows_per_tile // 5
    ngroups = chunks_per_tile // nbuf

    def zfill(i, _):
        for p in range(dh // 16):
            zbuf[i, pl.ds(p * 16, 16)] = jnp.zeros((16,), jnp.float32)
        return 0

    pltpu.sync_copy(row_ref.at[wid], ridx)
    pltpu.sync_copy(col_ref.at[wid], cidx)

    for y_ref, out_ref in ((y0_ref, out0_ref), (y1_ref, out1_ref)):
        lax.fori_loop(0, zrows, zfill, 0)
        for jj in range(5):
            pltpu.sync_copy(
                zbuf, acc.at[pl.ds(s * rows_per_tile + jj * zrows, zrows)])
        plsc.subcore_barrier()

        for b in range(nbuf):
            pltpu.async_copy(y_ref.at[ridx.at[b]], rows[b], gsem[b])

        def group(k, _):
            for b in range(nbuf):
                jj = nbuf * k + b
                pltpu.make_async_copy(y_ref.at[ridx.at[jj]], rows[b],
                                      gsem[b]).wait()
                pltpu.async_copy(rows[b], acc.at[cidx.at[jj]], ssem[b],
                                 add=True)
            for b in range(nbuf):
                jj = nbuf * k + b
                pltpu.make_async_copy(rows[b], acc.at[cidx.at[0]],
                                      ssem[b]).wait()
                pltpu.async_copy(y_ref.at[ridx.at[jj + nbuf]], rows[b],
                                 gsem[b])
            return 0
        lax.fori_loop(0, ngroups - 1, group, 0)

        for b in range(nbuf):
            jj = nbuf * (ngroups - 1) + b
            pltpu.make_async_copy(y_ref.at[ridx.at[jj]], rows[b],
                                  gsem[b]).wait()
            pltpu.async_copy(rows[b], acc.at[cidx.at[jj]], ssem[b], add=True)
        for b in range(nbuf):
            pltpu.make_async_copy(rows[b], acc.at[cidx.at[0]], ssem[b]).wait()

        plsc.subcore_barrier()
        for jj in range(5):
            base = s * rows_per_tile + jj * zrows
            pltpu.sync_copy(acc.at[pl.ds(base, zrows)], zbuf)
            pltpu.sync_copy(zbuf, out_ref.at[c, pl.ds(base, zrows)])


def _prescale_body(rb, dh, degp_ref, x_ref, y0_ref, y1_ref):
    i = pl.program_id(0)
    deg = (degp_ref[0, pl.ds(i * rb, rb), 0:1]
           + degp_ref[1, pl.ds(i * rb, rb), 0:1] + 1.0)
    dinv = lax.rsqrt(deg)
    y0_ref[...] = x_ref[:, pl.ds(0, dh)] * dinv
    y1_ref[...] = x_ref[:, pl.ds(dh, dh)] * dinv


def _final_body(rb, s00_ref, s01_ref, s10_ref, s11_ref, y0_ref, y1_ref,
                degp_ref, wt_ref, b_ref, g_ref, beta_ref, o_ref):
    i = pl.program_id(0)
    deg = (degp_ref[0, pl.ds(i * rb, rb), 0:1]
           + degp_ref[1, pl.ds(i * rb, rb), 0:1] + 1.0)
    dinv = lax.rsqrt(deg)
    dh = y0_ref.shape[-1]
    agg_l = (s00_ref[0] + s10_ref[0] + y0_ref[...]) * dinv
    agg_r = (s01_ref[0] + s11_ref[0] + y1_ref[...]) * dinv
    h = jnp.dot(agg_l, wt_ref[pl.ds(0, dh), :],
                preferred_element_type=jnp.float32)
    h = h + jnp.dot(agg_r, wt_ref[pl.ds(dh, dh), :],
                    preferred_element_type=jnp.float32)
    h = h + b_ref[...]
    mean = jnp.mean(h, axis=1, keepdims=True)
    zc = h - mean
    var = jnp.mean(zc * zc, axis=1, keepdims=True)
    o_ref[...] = zc * lax.rsqrt(var + 1e-5) * g_ref[...] + beta_ref[...]


@jax.jit
def kernel(x, edge_index, W, b, gamma, beta):
    n, d = x.shape
    dh = d // 2
    e = edge_index.shape[1]
    nchunks = e // CHUNK
    chunks_per_tile = nchunks // NW

    row3d = edge_index[0].astype(jnp.int32).reshape(NW, chunks_per_tile, CHUNK)
    col3d = edge_index[1].astype(jnp.int32).reshape(NW, chunks_per_tile, CHUNK)

    mesh = plsc.VectorSubcoreMesh(
        core_axis_name="c", subcore_axis_name="s",
        num_cores=NC, num_subcores=NS)

    n_pad = ((n + NS * 40 - 1) // (NS * 40)) * (NS * 40)
    rows_per_tile = n_pad // NS

    deg_call = pl.kernel(
        functools.partial(_deg_body, n_pad, chunks_per_tile),
        out_type=jax.ShapeDtypeStruct((NC, n_pad, 16), jnp.float32),
        mesh=mesh,
        scratch_types=[
            pltpu.VMEM((chunks_per_tile, CHUNK), jnp.int32),
            pltpu.VMEM((CHUNK, 16), jnp.float32),
            pltpu.VMEM((rows_per_tile, 16), jnp.float32),
            pltpu.VMEM_SHARED((n_pad, 16), jnp.float32),
            pltpu.SemaphoreType.DMA,
        ],
        compiler_params=pltpu.CompilerParams(use_tc_tiling_on_sc=False),
    )
    degp = deg_call(col3d)

    nb = 10
    rb = n // nb
    y0, y1 = pl.pallas_call(
        functools.partial(_prescale_body, rb, dh),
        grid=(nb,),
        in_specs=[
            pl.BlockSpec((NC, n_pad, 16), lambda i: (0, 0, 0)),
            pl.BlockSpec((rb, d), lambda i: (i, 0)),
        ],
        out_specs=[
            pl.BlockSpec((rb, dh), lambda i: (i, 0)),
            pl.BlockSpec((rb, dh), lambda i: (i, 0)),
        ],
        out_shape=[
            jax.ShapeDtypeStruct((n, dh), jnp.float32),
            jax.ShapeDtypeStruct((n, dh), jnp.float32),
        ],
    )(degp, x)

    nbuf = 5
    agg_call = pl.kernel(
        functools.partial(_agg_body, n_pad, dh, chunks_per_tile, nbuf),
        out_type=[
            jax.ShapeDtypeStruct((NC, n_pad, dh), jnp.float32),
            jax.ShapeDtypeStruct((NC, n_pad, dh), jnp.float32),
        ],
        mesh=mesh,
        scratch_types=(
            [pltpu.VMEM((chunks_per_tile, CHUNK), jnp.int32),
             pltpu.VMEM((chunks_per_tile, CHUNK), jnp.int32)]
            + [pltpu.VMEM((CHUNK, dh), jnp.float32) for _ in range(nbuf)]
            + [pltpu.VMEM((rows_per_tile // 5, dh), jnp.float32),
               pltpu.VMEM_SHARED((n_pad, dh), jnp.float32)]
            + [pltpu.SemaphoreType.DMA for _ in range(2 * nbuf)]
        ),
        compiler_params=pltpu.CompilerParams(use_tc_tiling_on_sc=False),
    )
    sp0, sp1 = agg_call(y0, y1, row3d, col3d)

    out = pl.pallas_call(
        functools.partial(_final_body, rb),
        grid=(nb,),
        in_specs=[
            pl.BlockSpec((1, rb, dh), lambda i: (0, i, 0)),
            pl.BlockSpec((1, rb, dh), lambda i: (0, i, 0)),
            pl.BlockSpec((1, rb, dh), lambda i: (1, i, 0)),
            pl.BlockSpec((1, rb, dh), lambda i: (1, i, 0)),
            pl.BlockSpec((rb, dh), lambda i: (i, 0)),
            pl.BlockSpec((rb, dh), lambda i: (i, 0)),
            pl.BlockSpec((NC, n_pad, 16), lambda i: (0, 0, 0)),
            pl.BlockSpec((d, d), lambda i: (0, 0)),
            pl.BlockSpec((1, d), lambda i: (0, 0)),
            pl.BlockSpec((1, d), lambda i: (0, 0)),
            pl.BlockSpec((1, d), lambda i: (0, 0)),
        ],
        out_specs=pl.BlockSpec((rb, d), lambda i: (i, 0)),
        out_shape=jax.ShapeDtypeStruct((n, d), jnp.float32),
    )(sp0, sp1, sp0, sp1,
      y0, y1, degp, W.T, b.reshape(1, d), gamma.reshape(1, d),
      beta.reshape(1, d))
    return out


# speedup vs baseline: 35.5566x; 1.0031x over previous
import functools

import jax
import jax.numpy as jnp
from jax import lax
from jax.experimental import pallas as pl
from jax.experimental.pallas import tpu as pltpu
from jax.experimental.pallas import tpu_sc as plsc

NC = 2
NS = 16
NW = NC * NS

CHUNK = 80


def _deg_body(n_pad, chunks_per_tile, col_ref, out_ref, cidx, ones_v, zbuf,
              acc, dsem):
    c = lax.axis_index("c")
    s = lax.axis_index("s")
    wid = c * NS + s
    rows_per_tile = n_pad // NS

    def fill(i, _):
        ones_v[i] = jnp.ones((16,), jnp.float32)
        return 0
    lax.fori_loop(0, CHUNK, fill, 0)

    def zfill(i, _):
        zbuf[i] = jnp.zeros((16,), jnp.float32)
        return 0
    lax.fori_loop(0, rows_per_tile, zfill, 0)

    pltpu.sync_copy(zbuf, acc.at[pl.ds(s * rows_per_tile, rows_per_tile)])
    plsc.subcore_barrier()

    pltpu.sync_copy(col_ref.at[wid], cidx)

    def step(j, _):
        pltpu.async_copy(ones_v, acc.at[cidx.at[j]], dsem, add=True)
        return 0
    lax.fori_loop(0, chunks_per_tile, step, 0)

    def drain(j, _):
        pltpu.make_async_copy(ones_v, acc.at[cidx.at[0]], dsem).wait()
        return 0
    lax.fori_loop(0, chunks_per_tile, drain, 0)

    plsc.subcore_barrier()
    pltpu.sync_copy(acc.at[pl.ds(s * rows_per_tile, rows_per_tile)], zbuf)
    pltpu.sync_copy(zbuf,
                    out_ref.at[c, pl.ds(s * rows_per_tile, rows_per_tile)])


def _agg_body(n_pad, dh, chunks_per_tile, nbuf, y0_ref, y1_ref, row_ref,
              col_ref, out0_ref, out1_ref, ridx, cidx, *scratch):
    rows = scratch[:nbuf]
    zbuf = scratch[nbuf]
    acc = scratch[nbuf + 1]
    gsem = scratch[nbuf + 2:2 * nbuf + 2]
    ssem = scratch[2 * nbuf + 2:]
    c = lax.axis_index("c")
    s = lax.axis_index("s")
    wid = c * NS + s
    rows_per_tile = n_pad // NS
    zrows = rows_per_tile // 5
    ngroups = chunks_per_tile // nbuf

    def zfill(i, _):
        for p in range(dh // 16):
            zbuf[i, pl.ds(p * 16, 16)] = jnp.zeros((16,), jnp.float32)
        return 0

    pltpu.sync_copy(row_ref.at[wid], ridx)
    pltpu.sync_copy(col_ref.at[wid], cidx)

    lax.fori_loop(0, zrows, zfill, 0)
    for y_ref, out_ref in ((y0_ref, out0_ref), (y1_ref, out1_ref)):
        for jj in range(5):
            pltpu.sync_copy(
                zbuf, acc.at[pl.ds(s * rows_per_tile + jj * zrows, zrows)])
        plsc.subcore_barrier()

        for b in range(nbuf):
            pltpu.async_copy(y_ref.at[ridx.at[b]], rows[b], gsem[b])

        def group(k, _):
            for b in range(nbuf):
                jj = nbuf * k + b
                pltpu.make_async_copy(y_ref.at[ridx.at[jj]], rows[b],
                                      gsem[b]).wait()
                pltpu.async_copy(rows[b], acc.at[cidx.at[jj]], ssem[b],
                                 add=True)
            for b in range(nbuf):
                jj = nbuf * k + b
                pltpu.make_async_copy(rows[b], acc.at[cidx.at[0]],
                                      ssem[b]).wait()
                pltpu.async_copy(y_ref.at[ridx.at[jj + nbuf]], rows[b],
                                 gsem[b])
            return 0
        lax.fori_loop(0, ngroups - 1, group, 0)

        for b in range(nbuf):
            jj = nbuf * (ngroups - 1) + b
            pltpu.make_async_copy(y_ref.at[ridx.at[jj]], rows[b],
                                  gsem[b]).wait()
            pltpu.async_copy(rows[b], acc.at[cidx.at[jj]], ssem[b], add=True)
        for b in range(nbuf):
            pltpu.make_async_copy(rows[b], acc.at[cidx.at[0]], ssem[b]).wait()

        plsc.subcore_barrier()
        base = s * rows_per_tile
        pltpu.sync_copy(acc.at[pl.ds(base, rows_per_tile)],
                        out_ref.at[c, pl.ds(base, rows_per_tile)])


def _prescale_body(rb, dh, degp_ref, x_ref, y0_ref, y1_ref):
    i = pl.program_id(0)
    deg = (degp_ref[0, pl.ds(i * rb, rb), 0:1]
           + degp_ref[1, pl.ds(i * rb, rb), 0:1] + 1.0)
    dinv = lax.rsqrt(deg)
    y0_ref[...] = x_ref[:, pl.ds(0, dh)] * dinv
    y1_ref[...] = x_ref[:, pl.ds(dh, dh)] * dinv


def _final_body(rb, s00_ref, s01_ref, s10_ref, s11_ref, y0_ref, y1_ref,
                degp_ref, wt_ref, b_ref, g_ref, beta_ref, o_ref):
    i = pl.program_id(0)
    deg = (degp_ref[0, pl.ds(i * rb, rb), 0:1]
           + degp_ref[1, pl.ds(i * rb, rb), 0:1] + 1.0)
    dinv = lax.rsqrt(deg)
    dh = y0_ref.shape[-1]
    agg_l = (s00_ref[0] + s10_ref[0] + y0_ref[...]) * dinv
    agg_r = (s01_ref[0] + s11_ref[0] + y1_ref[...]) * dinv
    h = jnp.dot(agg_l, wt_ref[pl.ds(0, dh), :],
                preferred_element_type=jnp.float32)
    h = h + jnp.dot(agg_r, wt_ref[pl.ds(dh, dh), :],
                    preferred_element_type=jnp.float32)
    h = h + b_ref[...]
    mean = jnp.mean(h, axis=1, keepdims=True)
    zc = h - mean
    var = jnp.mean(zc * zc, axis=1, keepdims=True)
    o_ref[...] = zc * lax.rsqrt(var + 1e-5) * g_ref[...] + beta_ref[...]


@jax.jit
def kernel(x, edge_index, W, b, gamma, beta):
    n, d = x.shape
    dh = d // 2
    e = edge_index.shape[1]
    nchunks = e // CHUNK
    chunks_per_tile = nchunks // NW

    row3d = edge_index[0].astype(jnp.int32).reshape(NW, chunks_per_tile, CHUNK)
    col3d = edge_index[1].astype(jnp.int32).reshape(NW, chunks_per_tile, CHUNK)

    mesh = plsc.VectorSubcoreMesh(
        core_axis_name="c", subcore_axis_name="s",
        num_cores=NC, num_subcores=NS)

    n_pad = ((n + NS * 40 - 1) // (NS * 40)) * (NS * 40)
    rows_per_tile = n_pad // NS

    deg_call = pl.kernel(
        functools.partial(_deg_body, n_pad, chunks_per_tile),
        out_type=jax.ShapeDtypeStruct((NC, n_pad, 16), jnp.float32),
        mesh=mesh,
        scratch_types=[
            pltpu.VMEM((chunks_per_tile, CHUNK), jnp.int32),
            pltpu.VMEM((CHUNK, 16), jnp.float32),
            pltpu.VMEM((rows_per_tile, 16), jnp.float32),
            pltpu.VMEM_SHARED((n_pad, 16), jnp.float32),
            pltpu.SemaphoreType.DMA,
        ],
        compiler_params=pltpu.CompilerParams(use_tc_tiling_on_sc=False),
    )
    degp = deg_call(col3d)

    nb = 10
    rb = n // nb
    y0, y1 = pl.pallas_call(
        functools.partial(_prescale_body, rb, dh),
        grid=(nb,),
        in_specs=[
            pl.BlockSpec((NC, n_pad, 16), lambda i: (0, 0, 0)),
            pl.BlockSpec((rb, d), lambda i: (i, 0)),
        ],
        out_specs=[
            pl.BlockSpec((rb, dh), lambda i: (i, 0)),
            pl.BlockSpec((rb, dh), lambda i: (i, 0)),
        ],
        out_shape=[
            jax.ShapeDtypeStruct((n, dh), jnp.float32),
            jax.ShapeDtypeStruct((n, dh), jnp.float32),
        ],
    )(degp, x)

    nbuf = 5
    agg_call = pl.kernel(
        functools.partial(_agg_body, n_pad, dh, chunks_per_tile, nbuf),
        out_type=[
            jax.ShapeDtypeStruct((NC, n_pad, dh), jnp.float32),
            jax.ShapeDtypeStruct((NC, n_pad, dh), jnp.float32),
        ],
        mesh=mesh,
        scratch_types=(
            [pltpu.VMEM((chunks_per_tile, CHUNK), jnp.int32),
             pltpu.VMEM((chunks_per_tile, CHUNK), jnp.int32)]
            + [pltpu.VMEM((CHUNK, dh), jnp.float32) for _ in range(nbuf)]
            + [pltpu.VMEM((rows_per_tile // 5, dh), jnp.float32),
               pltpu.VMEM_SHARED((n_pad, dh), jnp.float32)]
            + [pltpu.SemaphoreType.DMA for _ in range(2 * nbuf)]
        ),
        compiler_params=pltpu.CompilerParams(use_tc_tiling_on_sc=False),
    )
    sp0, sp1 = agg_call(y0, y1, row3d, col3d)

    out = pl.pallas_call(
        functools.partial(_final_body, rb),
        grid=(nb,),
        in_specs=[
            pl.BlockSpec((1, rb, dh), lambda i: (0, i, 0)),
            pl.BlockSpec((1, rb, dh), lambda i: (0, i, 0)),
            pl.BlockSpec((1, rb, dh), lambda i: (1, i, 0)),
            pl.BlockSpec((1, rb, dh), lambda i: (1, i, 0)),
            pl.BlockSpec((rb, dh), lambda i: (i, 0)),
            pl.BlockSpec((rb, dh), lambda i: (i, 0)),
            pl.BlockSpec((NC, n_pad, 16), lambda i: (0, 0, 0)),
            pl.BlockSpec((d, d), lambda i: (0, 0)),
            pl.BlockSpec((1, d), lambda i: (0, 0)),
            pl.BlockSpec((1, d), lambda i: (0, 0)),
            pl.BlockSpec((1, d), lambda i: (0, 0)),
        ],
        out_specs=pl.BlockSpec((rb, d), lambda i: (i, 0)),
        out_shape=jax.ShapeDtypeStruct((n, d), jnp.float32),
    )(sp0, sp1, sp0, sp1,
      y0, y1, degp, W.T, b.reshape(1, d), gamma.reshape(1, d),
      beta.reshape(1, d))
    return out


# speedup vs baseline: 37.3530x; 1.0505x over previous
import functools

import jax
import jax.numpy as jnp
from jax import lax
from jax.experimental import pallas as pl
from jax.experimental.pallas import tpu as pltpu
from jax.experimental.pallas import tpu_sc as plsc

NC = 2
NS = 16
NW = NC * NS

CHUNK = 80


def _deg_body(n_pad, chunks_per_tile, ei_ref, out_ref, cidx, ones_v, zbuf,
              acc, dsem):
    c = lax.axis_index("c")
    s = lax.axis_index("s")
    wid = c * NS + s
    rows_per_tile = n_pad // NS

    def fill(i, _):
        ones_v[i] = jnp.ones((16,), jnp.float32)
        return 0
    lax.fori_loop(0, CHUNK, fill, 0)

    def zfill(i, _):
        zbuf[i] = jnp.zeros((16,), jnp.float32)
        return 0
    lax.fori_loop(0, rows_per_tile, zfill, 0)

    pltpu.sync_copy(zbuf, acc.at[pl.ds(s * rows_per_tile, rows_per_tile)])
    plsc.subcore_barrier()

    ept = chunks_per_tile * CHUNK
    pltpu.sync_copy(ei_ref.at[1, pl.ds(wid * ept, ept)], cidx)

    def step(j, _):
        pltpu.async_copy(ones_v, acc.at[cidx.at[pl.ds(j * CHUNK, CHUNK)]],
                         dsem, add=True)
        return 0
    lax.fori_loop(0, chunks_per_tile, step, 0)

    def drain(j, _):
        pltpu.make_async_copy(ones_v, acc.at[cidx.at[pl.ds(0, CHUNK)]],
                              dsem).wait()
        return 0
    lax.fori_loop(0, chunks_per_tile, drain, 0)

    plsc.subcore_barrier()
    pltpu.sync_copy(acc.at[pl.ds(s * rows_per_tile, rows_per_tile)], zbuf)
    pltpu.sync_copy(zbuf,
                    out_ref.at[c, pl.ds(s * rows_per_tile, rows_per_tile)])


def _agg_body(n_pad, dh, chunks_per_tile, nbuf, y0_ref, y1_ref, ei_ref,
              out0_ref, out1_ref, ridx, cidx, *scratch):
    rows = scratch[:nbuf]
    zbuf = scratch[nbuf]
    acc = scratch[nbuf + 1]
    gsem = scratch[nbuf + 2:2 * nbuf + 2]
    ssem = scratch[2 * nbuf + 2:]
    c = lax.axis_index("c")
    s = lax.axis_index("s")
    wid = c * NS + s
    rows_per_tile = n_pad // NS
    zrows = rows_per_tile // 5
    ngroups = chunks_per_tile // nbuf

    def zfill(i, _):
        for p in range(dh // 16):
            zbuf[i, pl.ds(p * 16, 16)] = jnp.zeros((16,), jnp.float32)
        return 0

    ept = chunks_per_tile * CHUNK
    pltpu.sync_copy(ei_ref.at[0, pl.ds(wid * ept, ept)], ridx)
    pltpu.sync_copy(ei_ref.at[1, pl.ds(wid * ept, ept)], cidx)

    def islice(ref, jj):
        return ref.at[pl.ds(pl.multiple_of(jj * CHUNK, CHUNK), CHUNK)]

    lax.fori_loop(0, zrows, zfill, 0)
    for y_ref, out_ref in ((y0_ref, out0_ref), (y1_ref, out1_ref)):
        for jj in range(5):
            pltpu.sync_copy(
                zbuf, acc.at[pl.ds(s * rows_per_tile + jj * zrows, zrows)])
        plsc.subcore_barrier()

        for b in range(nbuf):
            pltpu.async_copy(y_ref.at[islice(ridx, b)], rows[b], gsem[b])

        def group(k, _):
            for b in range(nbuf):
                jj = nbuf * k + b
                pltpu.make_async_copy(y_ref.at[islice(ridx, jj)], rows[b],
                                      gsem[b]).wait()
                pltpu.async_copy(rows[b], acc.at[islice(cidx, jj)], ssem[b],
                                 add=True)
            for b in range(nbuf):
                jj = nbuf * k + b
                pltpu.make_async_copy(rows[b], acc.at[islice(cidx, 0)],
                                      ssem[b]).wait()
                pltpu.async_copy(y_ref.at[islice(ridx, jj + nbuf)], rows[b],
                                 gsem[b])
            return 0
        lax.fori_loop(0, ngroups - 1, group, 0)

        for b in range(nbuf):
            jj = nbuf * (ngroups - 1) + b
            pltpu.make_async_copy(y_ref.at[islice(ridx, jj)], rows[b],
                                  gsem[b]).wait()
            pltpu.async_copy(rows[b], acc.at[islice(cidx, jj)], ssem[b],
                             add=True)
        for b in range(nbuf):
            pltpu.make_async_copy(rows[b], acc.at[islice(cidx, 0)],
                                  ssem[b]).wait()

        plsc.subcore_barrier()
        base = s * rows_per_tile
        pltpu.sync_copy(acc.at[pl.ds(base, rows_per_tile)],
                        out_ref.at[c, pl.ds(base, rows_per_tile)])


def _prescale_body(rb, dh, degp_ref, x_ref, y0_ref, y1_ref):
    i = pl.program_id(0)
    deg = (degp_ref[0, pl.ds(i * rb, rb), 0:1]
           + degp_ref[1, pl.ds(i * rb, rb), 0:1] + 1.0)
    dinv = lax.rsqrt(deg)
    y0_ref[...] = x_ref[:, pl.ds(0, dh)] * dinv
    y1_ref[...] = x_ref[:, pl.ds(dh, dh)] * dinv


def _final_body(rb, s00_ref, s01_ref, s10_ref, s11_ref, y0_ref, y1_ref,
                degp_ref, wt_ref, b_ref, g_ref, beta_ref, o_ref):
    i = pl.program_id(0)
    deg = (degp_ref[0, pl.ds(i * rb, rb), 0:1]
           + degp_ref[1, pl.ds(i * rb, rb), 0:1] + 1.0)
    dinv = lax.rsqrt(deg)
    dh = y0_ref.shape[-1]
    agg_l = (s00_ref[0] + s10_ref[0] + y0_ref[...]) * dinv
    agg_r = (s01_ref[0] + s11_ref[0] + y1_ref[...]) * dinv
    h = jnp.dot(agg_l, wt_ref[pl.ds(0, dh), :],
                preferred_element_type=jnp.float32)
    h = h + jnp.dot(agg_r, wt_ref[pl.ds(dh, dh), :],
                    preferred_element_type=jnp.float32)
    h = h + b_ref[...]
    mean = jnp.mean(h, axis=1, keepdims=True)
    zc = h - mean
    var = jnp.mean(zc * zc, axis=1, keepdims=True)
    o_ref[...] = zc * lax.rsqrt(var + 1e-5) * g_ref[...] + beta_ref[...]


@jax.jit
def kernel(x, edge_index, W, b, gamma, beta):
    n, d = x.shape
    dh = d // 2
    e = edge_index.shape[1]
    nchunks = e // CHUNK
    chunks_per_tile = nchunks // NW

    ei32 = edge_index.astype(jnp.int32)

    mesh = plsc.VectorSubcoreMesh(
        core_axis_name="c", subcore_axis_name="s",
        num_cores=NC, num_subcores=NS)

    n_pad = ((n + NS * 40 - 1) // (NS * 40)) * (NS * 40)
    rows_per_tile = n_pad // NS

    deg_call = pl.kernel(
        functools.partial(_deg_body, n_pad, chunks_per_tile),
        out_type=jax.ShapeDtypeStruct((NC, n_pad, 16), jnp.float32),
        mesh=mesh,
        scratch_types=[
            pltpu.VMEM((chunks_per_tile * CHUNK,), jnp.int32),
            pltpu.VMEM((CHUNK, 16), jnp.float32),
            pltpu.VMEM((rows_per_tile, 16), jnp.float32),
            pltpu.VMEM_SHARED((n_pad, 16), jnp.float32),
            pltpu.SemaphoreType.DMA,
        ],
        compiler_params=pltpu.CompilerParams(use_tc_tiling_on_sc=False),
    )
    degp = deg_call(ei32)

    nb = 10
    rb = n // nb
    y0, y1 = pl.pallas_call(
        functools.partial(_prescale_body, rb, dh),
        grid=(nb,),
        in_specs=[
            pl.BlockSpec((NC, n_pad, 16), lambda i: (0, 0, 0)),
            pl.BlockSpec((rb, d), lambda i: (i, 0)),
        ],
        out_specs=[
            pl.BlockSpec((rb, dh), lambda i: (i, 0)),
            pl.BlockSpec((rb, dh), lambda i: (i, 0)),
        ],
        out_shape=[
            jax.ShapeDtypeStruct((n, dh), jnp.float32),
            jax.ShapeDtypeStruct((n, dh), jnp.float32),
        ],
    )(degp, x)

    nbuf = 5
    agg_call = pl.kernel(
        functools.partial(_agg_body, n_pad, dh, chunks_per_tile, nbuf),
        out_type=[
            jax.ShapeDtypeStruct((NC, n_pad, dh), jnp.float32),
            jax.ShapeDtypeStruct((NC, n_pad, dh), jnp.float32),
        ],
        mesh=mesh,
        scratch_types=(
            [pltpu.VMEM((chunks_per_tile * CHUNK,), jnp.int32),
             pltpu.VMEM((chunks_per_tile * CHUNK,), jnp.int32)]
            + [pltpu.VMEM((CHUNK, dh), jnp.float32) for _ in range(nbuf)]
            + [pltpu.VMEM((rows_per_tile // 5, dh), jnp.float32),
               pltpu.VMEM_SHARED((n_pad, dh), jnp.float32)]
            + [pltpu.SemaphoreType.DMA for _ in range(2 * nbuf)]
        ),
        compiler_params=pltpu.CompilerParams(use_tc_tiling_on_sc=False),
    )
    sp0, sp1 = agg_call(y0, y1, ei32)

    out = pl.pallas_call(
        functools.partial(_final_body, rb),
        grid=(nb,),
        in_specs=[
            pl.BlockSpec((1, rb, dh), lambda i: (0, i, 0)),
            pl.BlockSpec((1, rb, dh), lambda i: (0, i, 0)),
            pl.BlockSpec((1, rb, dh), lambda i: (1, i, 0)),
            pl.BlockSpec((1, rb, dh), lambda i: (1, i, 0)),
            pl.BlockSpec((rb, dh), lambda i: (i, 0)),
            pl.BlockSpec((rb, dh), lambda i: (i, 0)),
            pl.BlockSpec((NC, n_pad, 16), lambda i: (0, 0, 0)),
            pl.BlockSpec((d, d), lambda i: (0, 0)),
            pl.BlockSpec((1, d), lambda i: (0, 0)),
            pl.BlockSpec((1, d), lambda i: (0, 0)),
            pl.BlockSpec((1, d), lambda i: (0, 0)),
        ],
        out_specs=pl.BlockSpec((rb, d), lambda i: (i, 0)),
        out_shape=jax.ShapeDtypeStruct((n, d), jnp.float32),
    )(sp0, sp1, sp0, sp1,
      y0, y1, degp, W.T, b.reshape(1, d), gamma.reshape(1, d),
      beta.reshape(1, d))
    return out
